# baseline (speedup 1.0000x reference)
import jax
import jax.numpy as jnp
from jax import lax
from jax.experimental import pallas as pl
from jax.experimental.pallas import tpu as pltpu
from jax.experimental.pallas import tpu_sc as plsc

D_VOCAB = 100000
D_MODEL = 768
BATCH = 4
SEQ_LEN = 8192

NC = 2
NS = 16
NW = NC * NS
B_TOTAL = BATCH * SEQ_LEN
B_PER_W = B_TOTAL // NW
CHUNK = 32
N_CHUNKS = B_PER_W // CHUNK
N_BUF = 4


def _embed_body(tokens_hbm, table_hbm, out_hbm, idx_v, *rest):
    bufs = rest[:N_BUF]
    gsems = rest[N_BUF:2 * N_BUF]
    wsems = rest[2 * N_BUF:3 * N_BUF]

    wid = lax.axis_index("s") * NC + lax.axis_index("c")
    w_per_row = SEQ_LEN // B_PER_W
    b_idx = wid // w_per_row
    col = (wid % w_per_row) * B_PER_W
    pltpu.sync_copy(tokens_hbm.at[b_idx, pl.ds(col, B_PER_W)], idx_v)

    def gather(g):
        return pltpu.async_copy(
            table_hbm.at[idx_v.at[pl.ds(g * CHUNK, CHUNK)]],
            bufs[g % N_BUF], gsems[g % N_BUF])

    def write(g):
        return pltpu.async_copy(
            bufs[g % N_BUF],
            out_hbm.at[b_idx, pl.ds(col + g * CHUNK, CHUNK)],
            wsems[g % N_BUF])

    gds = [None] * N_CHUNKS
    wds = [None] * N_CHUNKS
    gds[0] = gather(0)
    gds[1] = gather(1)
    for g in range(N_CHUNKS):
        if g + 2 < N_CHUNKS:
            if g >= 2:
                wds[g - 2].wait()
            gds[g + 2] = gather(g + 2)
        gds[g].wait()
        wds[g] = write(g)
    for g in range(N_CHUNKS - 4, N_CHUNKS):
        wds[g].wait()


@jax.jit
def _embed(tokens, table):
    mesh = plsc.VectorSubcoreMesh(core_axis_name="c", subcore_axis_name="s")
    return pl.kernel(
        _embed_body,
        out_type=jax.ShapeDtypeStruct((BATCH, SEQ_LEN, D_MODEL), jnp.float32),
        mesh=mesh,
        scratch_types=(
            [pltpu.VMEM((B_PER_W,), jnp.int32)]
            + [pltpu.VMEM((CHUNK, D_MODEL), jnp.float32)] * N_BUF
            + [pltpu.SemaphoreType.DMA] * (2 * N_BUF)
        ),
    )(tokens, table)


def kernel(tokens, token_to_embed_map):
    return _embed(tokens.astype(jnp.int32), token_to_embed_map)
